# baseline (speedup 1.0000x reference)
import jax
import jax.numpy as jnp
from jax.experimental import pallas as pl

ALPHA = 0.2
D = 64
BLK = 2048


def _elu(v):
    return jnp.where(v > 0, v, jnp.exp(jnp.minimum(v, 0.0)) - 1.0)


def _lrelu(v):
    return jnp.where(v > 0, v, ALPHA * v)


def _cdiv(a, b):
    return (a + b - 1) // b


def _prep_body(x_ref, w1_ref, a1_ref, w2_ref, a2_ref,
               h1_ref, f11_ref, f21_ref, h2_ref, f12_ref, f22_ref):
    x = x_ref[...]
    for wr, ar, hr, f1r, f2r in ((w1_ref, a1_ref, h1_ref, f11_ref, f21_ref),
                                 (w2_ref, a2_ref, h2_ref, f12_ref, f22_ref)):
        h = jnp.dot(x, wr[...], preferred_element_type=jnp.float32)
        hr[...] = h
        av = ar[...]
        f1r[...] = jnp.sum(h * av[:, :D], axis=1, keepdims=True)
        f2r[...] = jnp.sum(h * av[:, D:], axis=1, keepdims=True)


def _tc_prep2(x, W1, a1, W2, a2):
    n, fin = x.shape
    grid = _cdiv(n, BLK)
    outs = []
    o_specs = []
    for _ in range(2):
        outs += [jax.ShapeDtypeStruct((n, D), jnp.float32),
                 jax.ShapeDtypeStruct((n, 1), jnp.float32),
                 jax.ShapeDtypeStruct((n, 1), jnp.float32)]
        o_specs += [pl.BlockSpec((BLK, D), lambda i: (i, 0)),
                    pl.BlockSpec((BLK, 1), lambda i: (i, 0)),
                    pl.BlockSpec((BLK, 1), lambda i: (i, 0))]
    return pl.pallas_call(
        _prep_body,
        grid=(grid,),
        in_specs=[pl.BlockSpec((BLK, fin), lambda i: (i, 0)),
                  pl.BlockSpec((fin, D), lambda i: (0, 0)),
                  pl.BlockSpec((1, 2 * D), lambda i: (0, 0)),
                  pl.BlockSpec((fin, D), lambda i: (0, 0)),
                  pl.BlockSpec((1, 2 * D), lambda i: (0, 0))],
        out_specs=o_specs,
        out_shape=outs,
    )(x, W1, a1, W2, a2)


def _epi_self_body(h1_ref, f11_ref, f21_ref, acc1_ref,
                   h2_ref, f12_ref, f22_ref, acc2_ref, o_ref):
    outs = []
    for hr, f1r, f2r, ar in ((h1_ref, f11_ref, f21_ref, acc1_ref),
                             (h2_ref, f12_ref, f22_ref, acc2_ref)):
        h = hr[...]
        wself = jnp.exp(-_lrelu(f1r[...] + f2r[...]))
        a = ar[...]
        num = wself * h + a[:, :D]
        den = wself + a[:, D:D + 1]
        outs.append(_elu(num / den))
    o_ref[...] = jnp.concatenate(outs, axis=1)


def _tc_epi_self2(h1, f11, f21, acc1, h2, f12, f22, acc2):
    n = h1.shape[0]
    grid = _cdiv(n, BLK)
    bs = [pl.BlockSpec((BLK, D), lambda i: (i, 0)),
          pl.BlockSpec((BLK, 1), lambda i: (i, 0)),
          pl.BlockSpec((BLK, 1), lambda i: (i, 0)),
          pl.BlockSpec((BLK, D + 1), lambda i: (i, 0))]
    return pl.pallas_call(
        _epi_self_body,
        grid=(grid,),
        in_specs=bs + bs,
        out_specs=pl.BlockSpec((BLK, 2 * D), lambda i: (i, 0)),
        out_shape=jax.ShapeDtypeStruct((n, 2 * D), jnp.float32),
    )(h1, f11, f21, acc1, h2, f12, f22, acc2)


def _epi_self1_body(h_ref, f1_ref, f2_ref, acc_ref, o_ref):
    h = h_ref[...]
    wself = jnp.exp(-_lrelu(f1_ref[...] + f2_ref[...]))
    a = acc_ref[...]
    o_ref[...] = _elu((wself * h + a[:, :D]) / (wself + a[:, D:D + 1]))


def _tc_epi_self1(h, f1, f2, acc):
    n = h.shape[0]
    grid = _cdiv(n, BLK)
    return pl.pallas_call(
        _epi_self1_body,
        grid=(grid,),
        in_specs=[pl.BlockSpec((BLK, D), lambda i: (i, 0)),
                  pl.BlockSpec((BLK, 1), lambda i: (i, 0)),
                  pl.BlockSpec((BLK, 1), lambda i: (i, 0)),
                  pl.BlockSpec((BLK, D + 1), lambda i: (i, 0))],
        out_specs=pl.BlockSpec((BLK, D), lambda i: (i, 0)),
        out_shape=jax.ShapeDtypeStruct((n, D), jnp.float32),
    )(h, f1, f2, acc)


def _prep1_body(x_ref, w_ref, a_ref, h_ref, f1_ref, f2_ref):
    x = x_ref[...]
    h = jnp.dot(x, w_ref[...], preferred_element_type=jnp.float32)
    h_ref[...] = h
    av = a_ref[...]
    f1_ref[...] = jnp.sum(h * av[:, :D], axis=1, keepdims=True)
    f2_ref[...] = jnp.sum(h * av[:, D:], axis=1, keepdims=True)


def _tc_prep1(x, W, a):
    n, fin = x.shape
    grid = _cdiv(n, BLK)
    return pl.pallas_call(
        _prep1_body,
        grid=(grid,),
        in_specs=[pl.BlockSpec((BLK, fin), lambda i: (i, 0)),
                  pl.BlockSpec((fin, D), lambda i: (0, 0)),
                  pl.BlockSpec((1, 2 * D), lambda i: (0, 0))],
        out_specs=[pl.BlockSpec((BLK, D), lambda i: (i, 0)),
                   pl.BlockSpec((BLK, 1), lambda i: (i, 0)),
                   pl.BlockSpec((BLK, 1), lambda i: (i, 0))],
        out_shape=[jax.ShapeDtypeStruct((n, D), jnp.float32),
                   jax.ShapeDtypeStruct((n, 1), jnp.float32),
                   jax.ShapeDtypeStruct((n, 1), jnp.float32)],
    )(x, W, a)


def _final_body(acc_ref, ho_ref, f1_ref, f2_ref, w_ref, b_ref, o_ref):
    wself = jnp.exp(-_lrelu(f1_ref[...] + f2_ref[...]))
    a = acc_ref[...]
    th = _elu((a[:, :D] + wself * ho_ref[...]) / (a[:, D:D + 1] + wself))
    logits = jnp.sum(th * w_ref[...], axis=1, keepdims=True) + b_ref[0, 0]
    o_ref[...] = 1.0 / (1.0 + jnp.exp(-logits))


def _tc_final(acc, ho, f1, f2, W_out, b_out):
    n = acc.shape[0]
    return pl.pallas_call(
        _final_body,
        grid=(1,),
        in_specs=[pl.BlockSpec((n, D + 1), lambda i: (0, 0)),
                  pl.BlockSpec((n, D), lambda i: (0, 0)),
                  pl.BlockSpec((n, 1), lambda i: (0, 0)),
                  pl.BlockSpec((n, 1), lambda i: (0, 0)),
                  pl.BlockSpec((1, D), lambda i: (0, 0)),
                  pl.BlockSpec((1, 1), lambda i: (0, 0))],
        out_specs=pl.BlockSpec((n, 1), lambda i: (0, 0)),
        out_shape=jax.ShapeDtypeStruct((n, 1), jnp.float32),
    )(acc, ho, f1, f2, W_out.reshape(1, D), b_out.reshape(1, 1))


def _edge_acc(h, f1, f2, srcs, dsts, n):
    w = jnp.exp(-_lrelu(f1[srcs, 0] + f2[dsts, 0]))
    payload = jnp.concatenate([w[:, None] * h[dsts], w[:, None]], axis=1)
    return jax.ops.segment_sum(payload, srcs, num_segments=n)


def kernel(repo, repo_users, users, user_edges, teams, team_users, params):
    n_users = users.shape[0]
    n_repo = n_users + 1

    x = jnp.concatenate([users, repo[None, :]], axis=0)
    srcs_r = repo_users.astype(jnp.int32)
    dsts_r = jnp.full_like(srcs_r, n_repo - 1)
    h1, f11, f21, h2, f12, f22 = _tc_prep2(
        x, params['W_repo_0'], params['a_repo_0'],
        params['W_repo_1'], params['a_repo_1'])
    acc1 = _edge_acc(h1, f11, f21, srcs_r, dsts_r, n_repo)
    acc2 = _edge_acc(h2, f12, f22, srcs_r, dsts_r, n_repo)
    xcat = _tc_epi_self2(h1, f11, f21, acc1, h2, f12, f22, acc2)
    ho, f1o, f2o = _tc_prep1(xcat, params['W_repo_out'], params['a_repo_out'])
    acco = _edge_acc(ho, f1o, f2o, srcs_r, dsts_r, n_repo)
    repo_h = _tc_epi_self1(ho, f1o, f2o, acco)[:n_users]

    e_rand = user_edges.shape[1] - n_users
    srcs_u = user_edges[0, :e_rand].astype(jnp.int32)
    dsts_u = user_edges[1, :e_rand].astype(jnp.int32)
    h1, f11, f21, h2, f12, f22 = _tc_prep2(
        repo_h, params['W_user_0'], params['a_user_0'],
        params['W_user_1'], params['a_user_1'])
    acc1 = _edge_acc(h1, f11, f21, srcs_u, dsts_u, n_users)
    acc2 = _edge_acc(h2, f12, f22, srcs_u, dsts_u, n_users)
    xcat = _tc_epi_self2(h1, f11, f21, acc1, h2, f12, f22, acc2)
    ho, f1o, f2o = _tc_prep1(xcat, params['W_user_out'], params['a_user_out'])
    acco = _edge_acc(ho, f1o, f2o, srcs_u, dsts_u, n_users)
    user_h = _tc_epi_self1(ho, f1o, f2o, acco)

    t_total = team_users.shape[0]
    nt = n_users + t_total
    x2 = jnp.concatenate([user_h, teams], axis=0)
    srcs_t = jnp.repeat(jnp.arange(t_total, dtype=jnp.int32) + n_users,
                        team_users.shape[1])
    dsts_t = team_users.reshape(-1).astype(jnp.int32)
    h1, f11, f21, h2, f12, f22 = _tc_prep2(
        x2, params['W_team_0'], params['a_team_0'],
        params['W_team_1'], params['a_team_1'])
    acc1 = _edge_acc(h1, f11, f21, srcs_t, dsts_t, nt)
    acc2 = _edge_acc(h2, f12, f22, srcs_t, dsts_t, nt)
    xcat = _tc_epi_self2(h1, f11, f21, acc1, h2, f12, f22, acc2)
    ho, f1o, f2o = _tc_prep1(xcat, params['W_team_out'], params['a_team_out'])
    acco_t = _edge_acc(ho, f1o, f2o, srcs_t, dsts_t, nt)[n_users:]
    return _tc_final(acco_t, ho[n_users:], f1o[n_users:], f2o[n_users:],
                     params['W_out'], params['b_out'])
